# baseline (speedup 1.0000x reference)
import jax
import jax.numpy as jnp
from jax import lax
from jax.experimental import pallas as pl
from jax.experimental.pallas import tpu as pltpu
from jax.experimental.pallas import tpu_sc as plsc

_NUM_TASKS = 8
_NUM_MODALITIES = 4
_PRIOR_LEN = 16
_EMBED_DIM = 768
_BATCH = 4096

_ROW = _PRIOR_LEN * _EMBED_DIM
_NROWS = 2 * _BATCH
_NC, _NS = 2, 16
_NW = _NC * _NS
_ROWS_PER_W = _NROWS // _NW
_CHUNK = 8
_NCHUNK = _ROWS_PER_W // _CHUNK


def _body(table_hbm, idx_hbm, out_hbm, idx_v, buf_v, sem):
    wid = lax.axis_index("s") * _NC + lax.axis_index("c")
    base = wid * _ROWS_PER_W
    pltpu.sync_copy(idx_hbm.at[wid], idx_v)

    @pl.loop(0, _NCHUNK)
    def _chunk_loop(j):
        pltpu.async_copy(table_hbm.at[idx_v.at[j]], buf_v, sem).wait()
        pltpu.sync_copy(buf_v, out_hbm.at[pl.ds(base + j * _CHUNK, _CHUNK)])


_sc_gather = pl.kernel(
    _body,
    out_type=jax.ShapeDtypeStruct((_NROWS, _ROW), jnp.float32),
    mesh=plsc.VectorSubcoreMesh(
        core_axis_name="c", subcore_axis_name="s",
        num_cores=_NC, num_subcores=_NS,
    ),
    scratch_types=[
        pltpu.VMEM((_NCHUNK, _CHUNK), jnp.int32),
        pltpu.VMEM((_CHUNK, _ROW), jnp.float32),
        pltpu.SemaphoreType.DMA,
    ],
)


def kernel(task_table, modality_table, task_idx, modality_idx):
    table = jnp.concatenate(
        [task_table.reshape(_NUM_TASKS, _ROW),
         modality_table.reshape(_NUM_MODALITIES, _ROW)], axis=0)
    idx = jnp.stack(
        [task_idx.astype(jnp.int32),
         modality_idx.astype(jnp.int32) + _NUM_TASKS], axis=1)
    idx = idx.reshape(_NW, _NCHUNK, _CHUNK)
    out = _sc_gather(table, idx)
    return out.reshape(_BATCH, 2 * _PRIOR_LEN, _EMBED_DIM)


# speedup vs baseline: 1.0584x; 1.0584x over previous
import jax
import jax.numpy as jnp
from jax import lax
from jax.experimental import pallas as pl
from jax.experimental.pallas import tpu as pltpu
from jax.experimental.pallas import tpu_sc as plsc

_NUM_TASKS = 8
_NUM_MODALITIES = 4
_PRIOR_LEN = 16
_EMBED_DIM = 768
_BATCH = 4096

_ROW = _PRIOR_LEN * _EMBED_DIM
_NROWS = 2 * _BATCH
_NC, _NS = 2, 16
_NW = _NC * _NS
_ROWS_PER_W = _NROWS // _NW
_CHUNK = 4
_NCHUNK = _ROWS_PER_W // _CHUNK


def _body(table_hbm, idx_hbm, out_hbm, idx_v, buf_a, buf_b,
          gsem_a, gsem_b, ssem_a, ssem_b):
    wid = lax.axis_index("s") * _NC + lax.axis_index("c")
    base = wid * _ROWS_PER_W
    pltpu.sync_copy(idx_hbm.at[wid], idx_v)

    def _gather(j, buf, sem):
        pltpu.async_copy(table_hbm.at[idx_v.at[j]], buf, sem)

    def _wait_gather(j, buf, sem):
        pltpu.make_async_copy(table_hbm.at[idx_v.at[j]], buf, sem).wait()

    def _scatter(j, buf, sem):
        pltpu.async_copy(
            buf, out_hbm.at[pl.ds(base + j * _CHUNK, _CHUNK)], sem)

    def _wait_scatter(buf, sem):
        pltpu.make_async_copy(buf, out_hbm.at[pl.ds(base, _CHUNK)], sem).wait()

    _gather(0, buf_a, gsem_a)

    @pl.loop(0, _NCHUNK // 2)
    def _pair(i):
        j0 = 2 * i
        j1 = j0 + 1

        @pl.when(i > 0)
        def _():
            _wait_scatter(buf_b, ssem_b)

        _gather(j1, buf_b, gsem_b)
        _wait_gather(j0, buf_a, gsem_a)
        _scatter(j0, buf_a, ssem_a)
        _wait_gather(j1, buf_b, gsem_b)
        _scatter(j1, buf_b, ssem_b)
        _wait_scatter(buf_a, ssem_a)

        @pl.when(j1 + 1 < _NCHUNK)
        def _():
            _gather(j1 + 1, buf_a, gsem_a)

    _wait_scatter(buf_b, ssem_b)


_sc_gather = pl.kernel(
    _body,
    out_type=jax.ShapeDtypeStruct((_NROWS, _ROW), jnp.float32),
    mesh=plsc.VectorSubcoreMesh(
        core_axis_name="c", subcore_axis_name="s",
        num_cores=_NC, num_subcores=_NS,
    ),
    scratch_types=[
        pltpu.VMEM((_NCHUNK, _CHUNK), jnp.int32),
        pltpu.VMEM((_CHUNK, _ROW), jnp.float32),
        pltpu.VMEM((_CHUNK, _ROW), jnp.float32),
        pltpu.SemaphoreType.DMA,
        pltpu.SemaphoreType.DMA,
        pltpu.SemaphoreType.DMA,
        pltpu.SemaphoreType.DMA,
    ],
)


def kernel(task_table, modality_table, task_idx, modality_idx):
    table = jnp.concatenate(
        [task_table.reshape(_NUM_TASKS, _ROW),
         modality_table.reshape(_NUM_MODALITIES, _ROW)], axis=0)
    idx = jnp.stack(
        [task_idx.astype(jnp.int32),
         modality_idx.astype(jnp.int32) + _NUM_TASKS], axis=1)
    idx = idx.reshape(_NW, _NCHUNK, _CHUNK)
    out = _sc_gather(table, idx)
    return out.reshape(_BATCH, 2 * _PRIOR_LEN, _EMBED_DIM)


# speedup vs baseline: 1.3782x; 1.3021x over previous
import jax
import jax.numpy as jnp
from jax import lax
from jax.experimental import pallas as pl
from jax.experimental.pallas import tpu as pltpu
from jax.experimental.pallas import tpu_sc as plsc

_NUM_TASKS = 8
_NUM_MODALITIES = 4
_PRIOR_LEN = 16
_EMBED_DIM = 768
_BATCH = 4096

_ROW = _PRIOR_LEN * _EMBED_DIM
_NC, _NS = 2, 16
_NW = _NC * _NS
_NG = _NW // 2
_B_PER_G = _BATCH // _NG
_K = 16


def _body(table_hbm, idx_hbm, out_hbm, tbl_v, idx_v, sem):
    wid = lax.axis_index("s") * _NC + lax.axis_index("c")
    half = wid % 2
    base = (wid // 2) * _B_PER_G
    pltpu.sync_copy(idx_hbm.at[wid], idx_v)

    @pl.when(half == 0)
    def _():
        pltpu.sync_copy(table_hbm.at[pl.ds(0, _NUM_TASKS)], tbl_v)

    @pl.when(half == 1)
    def _():
        pltpu.sync_copy(table_hbm.at[pl.ds(_NUM_TASKS, _NUM_MODALITIES)],
                        tbl_v.at[pl.ds(0, _NUM_MODALITIES)])

    @pl.loop(0, _B_PER_G, step=_K)
    def _block(i0):
        rows = idx_v[pl.ds(i0, _K)]
        for k in range(_K):
            pltpu.async_copy(
                tbl_v.at[rows[k]], out_hbm.at[base + i0 + k, half], sem)
        for _ in range(_K):
            pltpu.make_async_copy(
                tbl_v.at[0], out_hbm.at[base, half], sem).wait()


_sc_gather = pl.kernel(
    _body,
    out_type=jax.ShapeDtypeStruct((_BATCH, 2, _ROW), jnp.float32),
    mesh=plsc.VectorSubcoreMesh(
        core_axis_name="c", subcore_axis_name="s",
        num_cores=_NC, num_subcores=_NS,
    ),
    scratch_types=[
        pltpu.VMEM((_NUM_TASKS, _ROW), jnp.float32),
        pltpu.VMEM((_B_PER_G,), jnp.int32),
        pltpu.SemaphoreType.DMA,
    ],
)


def kernel(task_table, modality_table, task_idx, modality_idx):
    table = jnp.concatenate(
        [task_table.reshape(_NUM_TASKS, _ROW),
         modality_table.reshape(_NUM_MODALITIES, _ROW)], axis=0)
    idx = jnp.stack(
        [task_idx.astype(jnp.int32).reshape(_NG, _B_PER_G),
         modality_idx.astype(jnp.int32).reshape(_NG, _B_PER_G)], axis=1)
    idx = idx.reshape(_NW, _B_PER_G)
    out = _sc_gather(table, idx)
    return out.reshape(_BATCH, 2 * _PRIOR_LEN, _EMBED_DIM)


# speedup vs baseline: 1.7096x; 1.2404x over previous
import jax
import jax.numpy as jnp
from jax import lax
from jax.experimental import pallas as pl
from jax.experimental.pallas import tpu as pltpu
from jax.experimental.pallas import tpu_sc as plsc

_NUM_TASKS = 8
_NUM_MODALITIES = 4
_PRIOR_LEN = 16
_EMBED_DIM = 768
_BATCH = 4096

_ROW = _PRIOR_LEN * _EMBED_DIM
_NROWS = 2 * _BATCH
_NC, _NS = 2, 16
_NW = _NC * _NS
_NG = _NW // 2
_B_PER_G = _BATCH // _NG
_K = 16


def _body(table_hbm, idx_hbm, out_hbm, tbl_v, idx_v, sem):
    wid = lax.axis_index("s") * _NC + lax.axis_index("c")
    half = wid % 2
    base = (wid // 2) * _B_PER_G
    pltpu.sync_copy(idx_hbm.at[wid], idx_v)

    @pl.when(half == 0)
    def _():
        pltpu.sync_copy(table_hbm.at[pl.ds(0, _NUM_TASKS * _ROW)], tbl_v)

    @pl.when(half == 1)
    def _():
        pltpu.sync_copy(
            table_hbm.at[pl.ds(_NUM_TASKS * _ROW, _NUM_MODALITIES * _ROW)],
            tbl_v.at[pl.ds(0, _NUM_MODALITIES * _ROW)])

    def _row_copy(i, r):
        p = (base + i) * 2 + half
        pltpu.async_copy(tbl_v.at[pl.ds(r * _ROW, _ROW)],
                         out_hbm.at[pl.ds(p * _ROW, _ROW)], sem)

    def _wait_row():
        pltpu.make_async_copy(tbl_v.at[pl.ds(0, _ROW)],
                              out_hbm.at[pl.ds(0, _ROW)], sem).wait()

    rows0 = idx_v[pl.ds(0, _K)]
    for k in range(_K):
        _row_copy(k, rows0[k])

    @pl.loop(_K, _B_PER_G, step=_K)
    def _block(i0):
        rows = idx_v[pl.ds(i0, _K)]
        for k in range(_K):
            _wait_row()
            _row_copy(i0 + k, rows[k])

    for _ in range(_K):
        _wait_row()


_sc_gather = pl.kernel(
    _body,
    out_type=jax.ShapeDtypeStruct((_NROWS * _ROW,), jnp.float32),
    mesh=plsc.VectorSubcoreMesh(
        core_axis_name="c", subcore_axis_name="s",
        num_cores=_NC, num_subcores=_NS,
    ),
    scratch_types=[
        pltpu.VMEM((_NUM_TASKS * _ROW,), jnp.float32),
        pltpu.VMEM((_B_PER_G,), jnp.int32),
        pltpu.SemaphoreType.DMA,
    ],
)


def kernel(task_table, modality_table, task_idx, modality_idx):
    table = jnp.concatenate(
        [task_table.reshape(_NUM_TASKS * _ROW),
         modality_table.reshape(_NUM_MODALITIES * _ROW)])
    idx = jnp.stack(
        [task_idx.astype(jnp.int32).reshape(_NG, _B_PER_G),
         modality_idx.astype(jnp.int32).reshape(_NG, _B_PER_G)], axis=1)
    idx = idx.reshape(_NW, _B_PER_G)
    out = _sc_gather(table, idx)
    return out.reshape(_BATCH, 2 * _PRIOR_LEN, _EMBED_DIM)


# speedup vs baseline: 1.7123x; 1.0016x over previous
import jax
import jax.numpy as jnp
from jax import lax
from jax.experimental import pallas as pl
from jax.experimental.pallas import tpu as pltpu
from jax.experimental.pallas import tpu_sc as plsc

_NUM_TASKS = 8
_NUM_MODALITIES = 4
_PRIOR_LEN = 16
_EMBED_DIM = 768
_BATCH = 4096

_ROW = _PRIOR_LEN * _EMBED_DIM
_NROWS = 2 * _BATCH
_NC, _NS = 2, 16
_NW = _NC * _NS
_NG = _NW // 2
_B_PER_G = _BATCH // _NG
_K = 32


def _body(table_hbm, idx_hbm, out_hbm, tbl_v, idx_v, sem):
    wid = lax.axis_index("s") * _NC + lax.axis_index("c")
    half = wid % 2
    base = (wid // 2) * _B_PER_G
    pltpu.sync_copy(idx_hbm.at[wid], idx_v)

    @pl.when(half == 0)
    def _():
        pltpu.sync_copy(table_hbm.at[pl.ds(0, _NUM_TASKS * _ROW)], tbl_v)

    @pl.when(half == 1)
    def _():
        pltpu.sync_copy(
            table_hbm.at[pl.ds(_NUM_TASKS * _ROW, _NUM_MODALITIES * _ROW)],
            tbl_v.at[pl.ds(0, _NUM_MODALITIES * _ROW)])

    def _row_copy(i, r):
        p = (base + i) * 2 + half
        pltpu.async_copy(tbl_v.at[pl.ds(r * _ROW, _ROW)],
                         out_hbm.at[pl.ds(p * _ROW, _ROW)], sem)

    def _wait_row():
        pltpu.make_async_copy(tbl_v.at[pl.ds(0, _ROW)],
                              out_hbm.at[pl.ds(0, _ROW)], sem).wait()

    for g in range(_K // 16):
        rows0 = idx_v[pl.ds(g * 16, 16)]
        for k in range(16):
            _row_copy(g * 16 + k, rows0[k])

    @pl.loop(_K, _B_PER_G, step=16)
    def _block(i0):
        rows = idx_v[pl.ds(i0, 16)]
        for k in range(16):
            _wait_row()
            _row_copy(i0 + k, rows[k])

    for _ in range(_K):
        _wait_row()


_sc_gather = pl.kernel(
    _body,
    out_type=jax.ShapeDtypeStruct((_NROWS * _ROW,), jnp.float32),
    mesh=plsc.VectorSubcoreMesh(
        core_axis_name="c", subcore_axis_name="s",
        num_cores=_NC, num_subcores=_NS,
    ),
    scratch_types=[
        pltpu.VMEM((_NUM_TASKS * _ROW,), jnp.float32),
        pltpu.VMEM((_B_PER_G,), jnp.int32),
        pltpu.SemaphoreType.DMA,
    ],
)


def kernel(task_table, modality_table, task_idx, modality_idx):
    table = jnp.concatenate(
        [task_table.reshape(_NUM_TASKS * _ROW),
         modality_table.reshape(_NUM_MODALITIES * _ROW)])
    idx = jnp.stack(
        [task_idx.astype(jnp.int32).reshape(_NG, _B_PER_G),
         modality_idx.astype(jnp.int32).reshape(_NG, _B_PER_G)], axis=1)
    idx = idx.reshape(_NW, _B_PER_G)
    out = _sc_gather(table, idx)
    return out.reshape(_BATCH, 2 * _PRIOR_LEN, _EMBED_DIM)
